# Initial kernel scaffold; baseline (speedup 1.0000x reference)
import jax
import jax.numpy as jnp
from jax.experimental import pallas as pl


def kernel(X, batch, Q, edge_index, edge_weight, edge_attr, ln_g, ln_b, mlp_w1, mlp_b1, mlp_w2, mlp_b2, ws1, bs1, ws2, bs2, ws3, bs3, wt0, wt1, wt2, wt3, wt4, wt5):
    raise NotImplementedError("write your pallas kernel here")


# speedup vs baseline: 15.9931x; 15.9931x over previous
import functools

import jax
import jax.numpy as jnp
from jax import lax
from jax.experimental import pallas as pl
from jax.experimental.pallas import tpu as pltpu
from jax.experimental.pallas import tpu_sc as plsc

N = 10000
E = 160000
H = 128
NUM_RBF = 32
QDIM = 16
NMOL = 256
CUTOFF_UPPER = 5.0

BN = 1000
BE = 4000
CHUNK = 80
ZROWS = 1000

_f32 = jnp.float32


def _silu(x):
  return x * jax.nn.sigmoid(x)


def _k_charge(x_ref, b_ref, lng_ref, lnb_ref, w1_ref, b1_ref, w2_ref, b2_ref,
              c0_ref, fu_ref, qu_ref, fuq_ref):
  x = x_ref[...]
  c = lambda k: x[:, 128 * k:128 * (k + 1)]
  tr = c(0) + c(4) + c(8)
  a01 = 0.5 * (c(1) - c(3))
  a02 = 0.5 * (c(2) - c(6))
  a12 = 0.5 * (c(5) - c(7))
  na = 2.0 * (a01 * a01 + a02 * a02 + a12 * a12)
  s00 = c(0) - tr / 3.0
  s11 = c(4) - tr / 3.0
  s22 = c(8) - tr / 3.0
  s01 = 0.5 * (c(1) + c(3))
  s02 = 0.5 * (c(2) + c(6))
  s12 = 0.5 * (c(5) + c(7))
  na_sum = jnp.sum(na, axis=-1, keepdims=True)
  del na_sum
  ns = (s00 * s00 + s11 * s11 + s22 * s22
        + 2.0 * (s01 * s01 + s02 * s02 + s12 * s12))
  _x = jnp.concatenate([tr, na, ns], axis=-1)
  mu = jnp.sum(_x, axis=-1, keepdims=True) / 384.0
  d = _x - mu
  var = jnp.sum(d * d, axis=-1, keepdims=True) / 384.0
  xn = d / jnp.sqrt(var + 1e-5) * lng_ref[...] + lnb_ref[...]
  h = _silu(jnp.dot(xn, w1_ref[...], preferred_element_type=_f32)
            + b1_ref[...])
  cf = jnp.dot(h, w2_ref[...], preferred_element_type=_f32) + b2_ref[...]
  c0 = cf[:, :QDIM]
  f = cf[:, QDIM:]
  fu = f * f
  c0_ref[...] = c0
  fu_ref[...] = fu
  seg = jax.lax.broadcasted_iota(jnp.int32, (BN, NMOL), 1)
  oh = (b_ref[...] == seg).astype(_f32)

  @pl.when(pl.program_id(0) == 0)
  def _():
    qu_ref[...] = jnp.zeros_like(qu_ref)
    fuq_ref[...] = jnp.zeros_like(fuq_ref)

  qu_ref[...] += jnp.dot(oh.T, c0, preferred_element_type=_f32)
  fuq_ref[...] += jnp.dot(oh.T, fu, preferred_element_type=_f32)


def _k_qeq(c0_ref, fu_ref, b_ref, q_ref, qu_ref, fuq_ref, out_ref):
  seg = jax.lax.broadcasted_iota(jnp.int32, (BN, NMOL), 1)
  oh = (b_ref[...] == seg).astype(_f32)
  qn = jnp.dot(oh, qu_ref[...], preferred_element_type=_f32)
  fn = jnp.dot(oh, fuq_ref[...], preferred_element_type=_f32) + 1e-6
  fu = fu_ref[...]
  dq = q_ref[...] - qn
  out_ref[...] = c0_ref[...] + (fu / fn) * dq


def _k_edge(ea_ref, ci_ref, cj_ref, ew_ref, w1_ref, b1_ref, w2_ref, b2_ref,
            w3_ref, b3_ref, o0_ref, o1_ref, o2_ref):
  xe = jnp.concatenate([ea_ref[...], ci_ref[...][:, :QDIM],
                        cj_ref[...][:, :QDIM]], axis=-1)
  xe = _silu(jnp.dot(xe, w1_ref[...], preferred_element_type=_f32)
             + b1_ref[...])
  xe = _silu(jnp.dot(xe, w2_ref[...], preferred_element_type=_f32)
             + b2_ref[...])
  xe = _silu(jnp.dot(xe, w3_ref[...], preferred_element_type=_f32)
             + b3_ref[...])
  r = ew_ref[...]
  cut = 0.5 * (jnp.cos(r * (jnp.pi / CUTOFF_UPPER)) + 1.0)
  cut = cut * (r < CUTOFF_UPPER).astype(_f32)
  xe = xe * cut
  o0_ref[...] = xe[:, :H]
  o1_ref[...] = xe[:, H:2 * H]
  o2_ref[...] = xe[:, 2 * H:]


def _k_pack(x_ref, w0_ref, w1_ref, w2_ref, *o_refs):
  x = x_ref[...]
  c = lambda k: x[:, 128 * k:128 * (k + 1)]
  tn = sum(c(k) * c(k) for k in range(9)) + 1.0
  xn = [c(k) / tn for k in range(9)]
  tr = xn[0] + xn[4] + xn[8]
  a01 = 0.5 * (xn[1] - xn[3])
  a02 = 0.5 * (xn[2] - xn[6])
  a12 = 0.5 * (xn[5] - xn[7])
  s00 = xn[0] - tr / 3.0
  s01 = 0.5 * (xn[1] + xn[3])
  s02 = 0.5 * (xn[2] + xn[6])
  s11 = xn[4] - tr / 3.0
  s12 = 0.5 * (xn[5] + xn[7])
  w0, w1, w2 = w0_ref[...], w1_ref[...], w2_ref[...]
  dot = lambda a, w: jnp.dot(a, w, preferred_element_type=_f32)
  outs = [dot(tr, w0),
          dot(a01, w1), dot(a02, w1), dot(a12, w1),
          dot(s00, w2), dot(s01, w2), dot(s02, w2),
          dot(s11, w2), dot(s12, w2)]
  for r, v in zip(o_refs, outs):
    r[...] = v


def _compose(i0, a01, a02, a12, s00, s01, s02, s11, s12):
  t = i0 / 3.0
  return [t + s00, s01 + a01, s02 + a02,
          s01 - a01, t + s11, s12 + a12,
          s02 - a02, s12 - a12, t - s00 - s11]


def _k_final(x_ref, w3_ref, w4_ref, w5_ref, *refs):
  p_refs = refs[:9]
  m_refs = refs[9:18]
  o_ref = refs[18]
  x = x_ref[...]
  c = lambda k: x[:, 128 * k:128 * (k + 1)]
  tn = sum(c(k) * c(k) for k in range(9)) + 1.0
  xn = [c(k) / tn for k in range(9)]
  y = _compose(*[r[...] for r in p_refs])
  m = _compose(*[r[...] for r in m_refs])
  idx = lambda i, j: 3 * i + j
  cm = []
  for i in range(3):
    for j in range(3):
      acc = None
      for k in range(3):
        t = y[idx(i, k)] * m[idx(k, j)] + m[idx(i, k)] * y[idx(k, j)]
        acc = t if acc is None else acc + t
      cm.append(acc)
  normp1 = sum(v * v for v in cm) + 1.0
  trc = cm[0] + cm[4] + cm[8]
  ic = trc / normp1
  ac01 = 0.5 * (cm[1] - cm[3]) / normp1
  ac02 = 0.5 * (cm[2] - cm[6]) / normp1
  ac12 = 0.5 * (cm[5] - cm[7]) / normp1
  sc00 = (cm[0] - trc / 3.0) / normp1
  sc01 = 0.5 * (cm[1] + cm[3]) / normp1
  sc02 = 0.5 * (cm[2] + cm[6]) / normp1
  sc11 = (cm[4] - trc / 3.0) / normp1
  sc12 = 0.5 * (cm[5] + cm[7]) / normp1
  w3, w4, w5 = w3_ref[...], w4_ref[...], w5_ref[...]
  dot = lambda a, w: jnp.dot(a, w, preferred_element_type=_f32)
  dx = _compose(dot(ic, w3),
                dot(ac01, w4), dot(ac02, w4), dot(ac12, w4),
                dot(sc00, w5), dot(sc01, w5), dot(sc02, w5),
                dot(sc11, w5), dot(sc12, w5))
  for i in range(3):
    for j in range(3):
      acc = xn[idx(i, j)] + dx[idx(i, j)]
      for k in range(3):
        acc = acc + dx[idx(i, k)] * dx[idx(k, j)]
      o_ref[:, 128 * idx(i, j):128 * (idx(i, j) + 1)] = acc


def _mesh():
  return plsc.VectorSubcoreMesh(core_axis_name="c", subcore_axis_name="s",
                                num_cores=2)


def _sc_gather_rows(table, idx):
  n_idx = idx.shape[0]
  per_w = n_idx // 32
  n_chunk = per_w // CHUNK

  @functools.partial(
      pl.kernel, mesh=_mesh(),
      out_type=jax.ShapeDtypeStruct((n_idx, H), _f32),
      scratch_types=[
          pltpu.VMEM((CHUNK,), jnp.int32),
          pltpu.VMEM((CHUNK, H), _f32),
          pltpu.SemaphoreType.DMA,
      ],
  )
  def k(table_hbm, idx_hbm, out_hbm, idx_v, rows_v, sem):
    wid = lax.axis_index("s") * 2 + lax.axis_index("c")
    base0 = wid * per_w

    def body(kk, carry):
      base = base0 + kk * CHUNK
      pltpu.sync_copy(idx_hbm.at[pl.ds(base, CHUNK)], idx_v)
      pltpu.async_copy(table_hbm.at[idx_v], rows_v, sem).wait()
      pltpu.sync_copy(rows_v, out_hbm.at[pl.ds(base, CHUNK)])
      return carry

    lax.fori_loop(0, n_chunk, body, 0)

  return k(table, idx)


def _sc_message(p_a, p_b, w_a, w_b, src, dst, zrows):
  per_t = E // 16
  n_chunk = per_t // CHUNK

  @functools.partial(
      pl.kernel, mesh=_mesh(),
      out_type=jax.ShapeDtypeStruct((2, N, H), _f32),
      scratch_types=[
          pltpu.VMEM((CHUNK,), jnp.int32),
          pltpu.VMEM((CHUNK,), jnp.int32),
          pltpu.VMEM((CHUNK, H), _f32),
          pltpu.VMEM((CHUNK, H), _f32),
          pltpu.VMEM_SHARED((N, H), _f32),
          pltpu.SemaphoreType.DMA,
      ],
  )
  def k(pa_hbm, pb_hbm, wa_hbm, wb_hbm, src_hbm, dst_hbm, z_hbm, out_hbm,
        si_v, di_v, rows_v, wv_v, acc_sh, sem):
    cid = lax.axis_index("c")
    sid = lax.axis_index("s")

    @pl.when(sid < 10)
    def _():
      pltpu.sync_copy(z_hbm, acc_sh.at[pl.ds(sid * ZROWS, ZROWS)])

    plsc.subcore_barrier()

    def run(p_hbm, w_hbm):
      def body(kk, carry):
        base = sid * per_t + kk * CHUNK
        pltpu.sync_copy(src_hbm.at[pl.ds(base, CHUNK)], si_v)
        pltpu.sync_copy(dst_hbm.at[pl.ds(base, CHUNK)], di_v)
        pltpu.sync_copy(w_hbm.at[pl.ds(base, CHUNK)], wv_v)
        pltpu.async_copy(p_hbm.at[si_v], rows_v, sem).wait()

        def mul(e, c2):
          for v in range(H // 16):
            sl = pl.ds(v * 16, 16)
            rows_v[e, sl] = rows_v[e, sl] * wv_v[e, sl]
          return c2

        lax.fori_loop(0, CHUNK, mul, 0)
        pltpu.sync_copy(rows_v, acc_sh.at[di_v], add=True)
        return carry

      lax.fori_loop(0, n_chunk, body, 0)
      plsc.subcore_barrier()

      @pl.when(sid < 10)
      def _():
        pltpu.sync_copy(acc_sh.at[pl.ds(sid * ZROWS, ZROWS)],
                        out_hbm.at[cid, pl.ds(sid * ZROWS, ZROWS)])

    @pl.when(cid == 0)
    def _():
      run(pa_hbm, wa_hbm)

    @pl.when(cid == 1)
    def _():
      run(pb_hbm, wb_hbm)

  return k(p_a, p_b, w_a, w_b, src, dst, zrows)


def _full(shape):
  return pl.BlockSpec(shape, lambda i: (0,) * len(shape))


def kernel(X, batch, Q, edge_index, edge_weight, edge_attr, ln_g, ln_b,
           mlp_w1, mlp_b1, mlp_w2, mlp_b2, ws1, bs1, ws2, bs2, ws3, bs3,
           wt0, wt1, wt2, wt3, wt4, wt5):
  X2 = X.reshape(N, 9 * H)
  batch2 = batch.astype(jnp.int32).reshape(N, 1)
  Q2 = Q.reshape(N, 1)
  ei = edge_index.astype(jnp.int32)
  dst, src = ei[0], ei[1]

  gn = N // BN
  nspec = pl.BlockSpec((BN, 9 * H), lambda i: (i, 0))
  bspec = pl.BlockSpec((BN, 1), lambda i: (i, 0))
  q16 = pl.BlockSpec((BN, QDIM), lambda i: (i, 0))

  c0, fu, qu, fuq = pl.pallas_call(
      _k_charge,
      grid=(gn,),
      in_specs=[nspec, bspec, _full((1, 3 * H)), _full((1, 3 * H)),
                _full((3 * H, H)), _full((1, H)),
                _full((H, 2 * QDIM)), _full((1, 2 * QDIM))],
      out_specs=[q16, q16, _full((NMOL, QDIM)), _full((NMOL, QDIM))],
      out_shape=[jax.ShapeDtypeStruct((N, QDIM), _f32),
                 jax.ShapeDtypeStruct((N, QDIM), _f32),
                 jax.ShapeDtypeStruct((NMOL, QDIM), _f32),
                 jax.ShapeDtypeStruct((NMOL, QDIM), _f32)],
  )(X2, batch2, ln_g.reshape(1, -1), ln_b.reshape(1, -1),
    mlp_w1, mlp_b1.reshape(1, -1), mlp_w2, mlp_b2.reshape(1, -1))

  charges = pl.pallas_call(
      _k_qeq,
      grid=(gn,),
      in_specs=[q16, q16, bspec, bspec,
                _full((NMOL, QDIM)), _full((NMOL, QDIM))],
      out_specs=q16,
      out_shape=jax.ShapeDtypeStruct((N, QDIM), _f32),
  )(c0, fu, batch2, Q2, qu, fuq)

  idx_all = jnp.concatenate([dst, src])
  cpad = jnp.pad(charges, ((0, 0), (0, H - QDIM)))
  cc = _sc_gather_rows(cpad, idx_all)

  ge = E // BE
  espec = lambda w: pl.BlockSpec((BE, w), lambda i: (i, 0))
  cjspec = pl.BlockSpec((BE, H), lambda i: (i + ge, 0))
  w0, w1, w2 = pl.pallas_call(
      _k_edge,
      grid=(ge,),
      in_specs=[espec(NUM_RBF), espec(H), cjspec, espec(1),
                _full((NUM_RBF + 2 * QDIM, H)), _full((1, H)),
                _full((H, 2 * H)), _full((1, 2 * H)),
                _full((2 * H, 3 * H)), _full((1, 3 * H))],
      out_specs=[espec(H), espec(H), espec(H)],
      out_shape=[jax.ShapeDtypeStruct((E, H), _f32)] * 3,
  )(edge_attr, cc, cc, edge_weight.reshape(E, 1),
    ws1, bs1.reshape(1, -1), ws2, bs2.reshape(1, -1), ws3, bs3.reshape(1, -1))

  hspec = pl.BlockSpec((BN, H), lambda i: (i, 0))
  p_list = pl.pallas_call(
      _k_pack,
      grid=(gn,),
      in_specs=[nspec, _full((H, H)), _full((H, H)), _full((H, H))],
      out_specs=[hspec] * 9,
      out_shape=[jax.ShapeDtypeStruct((N, H), _f32)] * 9,
  )(X2, wt0, wt1, wt2)

  wsel = [w0, w1, w1, w1, w2, w2, w2, w2, w2]
  zrows = jnp.zeros((ZROWS, H), _f32)
  m_list = [None] * 9
  for ca in range(0, 9, 2):
    cb = min(ca + 1, 8)
    out2 = _sc_message(p_list[ca], p_list[cb], wsel[ca], wsel[cb],
                       src, dst, zrows)
    m_list[ca] = out2[0]
    if cb != ca:
      m_list[cb] = out2[1]

  xo2 = pl.pallas_call(
      _k_final,
      grid=(gn,),
      in_specs=[nspec, _full((H, H)), _full((H, H)), _full((H, H))]
               + [hspec] * 18,
      out_specs=nspec,
      out_shape=jax.ShapeDtypeStruct((N, 9 * H), _f32),
  )(X2, wt3, wt4, wt5, *p_list, *m_list)

  return xo2.reshape(N, 3, 3, H)


# speedup vs baseline: 27.3322x; 1.7090x over previous
import functools

import jax
import jax.numpy as jnp
from jax import lax
from jax.experimental import pallas as pl
from jax.experimental.pallas import tpu as pltpu
from jax.experimental.pallas import tpu_sc as plsc

N = 10000
E = 160000
H = 128
NUM_RBF = 32
QDIM = 16
NMOL = 256
CUTOFF_UPPER = 5.0

BN = 1000
BE = 4000
CHUNK = 80
ZROWS = 1000

_f32 = jnp.float32


def _silu(x):
  return x * jax.nn.sigmoid(x)


def _k_charge(x_ref, b_ref, lng_ref, lnb_ref, w1_ref, b1_ref, w2_ref, b2_ref,
              c0_ref, fu_ref, qu_ref, fuq_ref):
  x = x_ref[...]
  c = lambda k: x[:, 128 * k:128 * (k + 1)]
  tr = c(0) + c(4) + c(8)
  a01 = 0.5 * (c(1) - c(3))
  a02 = 0.5 * (c(2) - c(6))
  a12 = 0.5 * (c(5) - c(7))
  na = 2.0 * (a01 * a01 + a02 * a02 + a12 * a12)
  s00 = c(0) - tr / 3.0
  s11 = c(4) - tr / 3.0
  s22 = c(8) - tr / 3.0
  s01 = 0.5 * (c(1) + c(3))
  s02 = 0.5 * (c(2) + c(6))
  s12 = 0.5 * (c(5) + c(7))
  na_sum = jnp.sum(na, axis=-1, keepdims=True)
  del na_sum
  ns = (s00 * s00 + s11 * s11 + s22 * s22
        + 2.0 * (s01 * s01 + s02 * s02 + s12 * s12))
  _x = jnp.concatenate([tr, na, ns], axis=-1)
  mu = jnp.sum(_x, axis=-1, keepdims=True) / 384.0
  d = _x - mu
  var = jnp.sum(d * d, axis=-1, keepdims=True) / 384.0
  xn = d / jnp.sqrt(var + 1e-5) * lng_ref[...] + lnb_ref[...]
  h = _silu(jnp.dot(xn, w1_ref[...], preferred_element_type=_f32)
            + b1_ref[...])
  cf = jnp.dot(h, w2_ref[...], preferred_element_type=_f32) + b2_ref[...]
  c0 = cf[:, :QDIM]
  f = cf[:, QDIM:]
  fu = f * f
  c0_ref[...] = c0
  fu_ref[...] = fu
  seg = jax.lax.broadcasted_iota(jnp.int32, (BN, NMOL), 1)
  oh = (b_ref[...] == seg).astype(_f32)

  @pl.when(pl.program_id(0) == 0)
  def _():
    qu_ref[...] = jnp.zeros_like(qu_ref)
    fuq_ref[...] = jnp.zeros_like(fuq_ref)

  qu_ref[...] += jnp.dot(oh.T, c0, preferred_element_type=_f32)
  fuq_ref[...] += jnp.dot(oh.T, fu, preferred_element_type=_f32)


def _k_qeq(c0_ref, fu_ref, b_ref, q_ref, qu_ref, fuq_ref, out_ref):
  seg = jax.lax.broadcasted_iota(jnp.int32, (BN, NMOL), 1)
  oh = (b_ref[...] == seg).astype(_f32)
  qn = jnp.dot(oh, qu_ref[...], preferred_element_type=_f32)
  fn = jnp.dot(oh, fuq_ref[...], preferred_element_type=_f32) + 1e-6
  fu = fu_ref[...]
  dq = q_ref[...] - qn
  out_ref[...] = c0_ref[...] + (fu / fn) * dq


def _k_edge(ea_ref, ci_ref, cj_ref, ew_ref, w1_ref, b1_ref, w2_ref, b2_ref,
            w3_ref, b3_ref, o0_ref, o1_ref, o2_ref):
  xe = jnp.concatenate([ea_ref[...], ci_ref[...][:, :QDIM],
                        cj_ref[...][:, :QDIM]], axis=-1)
  xe = _silu(jnp.dot(xe, w1_ref[...], preferred_element_type=_f32)
             + b1_ref[...])
  xe = _silu(jnp.dot(xe, w2_ref[...], preferred_element_type=_f32)
             + b2_ref[...])
  xe = _silu(jnp.dot(xe, w3_ref[...], preferred_element_type=_f32)
             + b3_ref[...])
  r = ew_ref[...]
  cut = 0.5 * (jnp.cos(r * (jnp.pi / CUTOFF_UPPER)) + 1.0)
  cut = cut * (r < CUTOFF_UPPER).astype(_f32)
  xe = xe * cut
  o0_ref[...] = xe[:, :H]
  o1_ref[...] = xe[:, H:2 * H]
  o2_ref[...] = xe[:, 2 * H:]


def _k_pack(x_ref, w0_ref, w1_ref, w2_ref, *o_refs):
  x = x_ref[...]
  c = lambda k: x[:, 128 * k:128 * (k + 1)]
  tn = sum(c(k) * c(k) for k in range(9)) + 1.0
  xn = [c(k) / tn for k in range(9)]
  tr = xn[0] + xn[4] + xn[8]
  a01 = 0.5 * (xn[1] - xn[3])
  a02 = 0.5 * (xn[2] - xn[6])
  a12 = 0.5 * (xn[5] - xn[7])
  s00 = xn[0] - tr / 3.0
  s01 = 0.5 * (xn[1] + xn[3])
  s02 = 0.5 * (xn[2] + xn[6])
  s11 = xn[4] - tr / 3.0
  s12 = 0.5 * (xn[5] + xn[7])
  w0, w1, w2 = w0_ref[...], w1_ref[...], w2_ref[...]
  dot = lambda a, w: jnp.dot(a, w, preferred_element_type=_f32)
  outs = [dot(tr, w0),
          dot(a01, w1), dot(a02, w1), dot(a12, w1),
          dot(s00, w2), dot(s01, w2), dot(s02, w2),
          dot(s11, w2), dot(s12, w2)]
  for r, v in zip(o_refs, outs):
    r[...] = v


def _compose(i0, a01, a02, a12, s00, s01, s02, s11, s12):
  t = i0 / 3.0
  return [t + s00, s01 + a01, s02 + a02,
          s01 - a01, t + s11, s12 + a12,
          s02 - a02, s12 - a12, t - s00 - s11]


def _k_final(x_ref, w3_ref, w4_ref, w5_ref, *refs):
  p_refs = refs[:9]
  m_refs = refs[9:18]
  o_ref = refs[18]
  x = x_ref[...]
  c = lambda k: x[:, 128 * k:128 * (k + 1)]
  tn = sum(c(k) * c(k) for k in range(9)) + 1.0
  xn = [c(k) / tn for k in range(9)]
  y = _compose(*[r[...] for r in p_refs])
  m = _compose(*[r[...] for r in m_refs])
  idx = lambda i, j: 3 * i + j
  cm = []
  for i in range(3):
    for j in range(3):
      acc = None
      for k in range(3):
        t = y[idx(i, k)] * m[idx(k, j)] + m[idx(i, k)] * y[idx(k, j)]
        acc = t if acc is None else acc + t
      cm.append(acc)
  normp1 = sum(v * v for v in cm) + 1.0
  trc = cm[0] + cm[4] + cm[8]
  ic = trc / normp1
  ac01 = 0.5 * (cm[1] - cm[3]) / normp1
  ac02 = 0.5 * (cm[2] - cm[6]) / normp1
  ac12 = 0.5 * (cm[5] - cm[7]) / normp1
  sc00 = (cm[0] - trc / 3.0) / normp1
  sc01 = 0.5 * (cm[1] + cm[3]) / normp1
  sc02 = 0.5 * (cm[2] + cm[6]) / normp1
  sc11 = (cm[4] - trc / 3.0) / normp1
  sc12 = 0.5 * (cm[5] + cm[7]) / normp1
  w3, w4, w5 = w3_ref[...], w4_ref[...], w5_ref[...]
  dot = lambda a, w: jnp.dot(a, w, preferred_element_type=_f32)
  dx = _compose(dot(ic, w3),
                dot(ac01, w4), dot(ac02, w4), dot(ac12, w4),
                dot(sc00, w5), dot(sc01, w5), dot(sc02, w5),
                dot(sc11, w5), dot(sc12, w5))
  for i in range(3):
    for j in range(3):
      acc = xn[idx(i, j)] + dx[idx(i, j)]
      for k in range(3):
        acc = acc + dx[idx(i, k)] * dx[idx(k, j)]
      o_ref[:, 128 * idx(i, j):128 * (idx(i, j) + 1)] = acc


def _mesh():
  return plsc.VectorSubcoreMesh(core_axis_name="c", subcore_axis_name="s",
                                num_cores=2)


def _sc_gather_rows(table, idx):
  n_idx = idx.shape[0]
  per_w = n_idx // 32
  n_chunk = per_w // CHUNK

  @functools.partial(
      pl.kernel, mesh=_mesh(),
      out_type=jax.ShapeDtypeStruct((n_idx, H), _f32),
      scratch_types=[
          pltpu.VMEM((CHUNK,), jnp.int32),
          pltpu.VMEM((CHUNK, H), _f32),
          pltpu.SemaphoreType.DMA,
      ],
  )
  def k(table_hbm, idx_hbm, out_hbm, idx_v, rows_v, sem):
    wid = lax.axis_index("s") * 2 + lax.axis_index("c")
    base0 = wid * per_w

    def body(kk, carry):
      base = base0 + kk * CHUNK
      pltpu.sync_copy(idx_hbm.at[pl.ds(base, CHUNK)], idx_v)
      pltpu.async_copy(table_hbm.at[idx_v], rows_v, sem).wait()
      pltpu.sync_copy(rows_v, out_hbm.at[pl.ds(base, CHUNK)])
      return carry

    lax.fori_loop(0, n_chunk, body, 0)

  return k(table, idx)


def _sc_message(p_a, p_b, w_a, w_b, src, dst, zrows):
  per_t = E // 16
  n_chunk = per_t // CHUNK

  @functools.partial(
      pl.kernel, mesh=_mesh(),
      out_type=jax.ShapeDtypeStruct((2, N, H), _f32),
      scratch_types=[
          pltpu.VMEM((2, CHUNK), jnp.int32),
          pltpu.VMEM((2, CHUNK), jnp.int32),
          pltpu.VMEM((CHUNK, H), _f32),
          pltpu.VMEM((CHUNK, H), _f32),
          pltpu.VMEM((CHUNK, H), _f32),
          pltpu.VMEM((CHUNK, H), _f32),
          pltpu.VMEM_SHARED((N, H), _f32),
          pltpu.SemaphoreType.DMA,
          pltpu.SemaphoreType.DMA,
          pltpu.SemaphoreType.DMA,
          pltpu.SemaphoreType.DMA,
          pltpu.SemaphoreType.DMA,
          pltpu.SemaphoreType.DMA,
      ],
  )
  def k(pa_hbm, pb_hbm, wa_hbm, wb_hbm, src_hbm, dst_hbm, z_hbm, out_hbm,
        si2, di2, rows0, rows1, wv0, wv1, acc_sh, g0, g1, s0, s1, d0, d1):
    cid = lax.axis_index("c")
    sid = lax.axis_index("s")
    rows = (rows0, rows1)
    wv = (wv0, wv1)
    gsem = (g0, g1)
    ssem = (s0, s1)
    dwsem = (d0, d1)

    @pl.when(sid < 10)
    def _():
      pltpu.sync_copy(z_hbm, acc_sh.at[pl.ds(sid * ZROWS, ZROWS)])

    plsc.subcore_barrier()

    def run(p_hbm, w_hbm):
      def fill_idx(slot, c):
        base = sid * per_t + c * CHUNK
        pltpu.async_copy(src_hbm.at[pl.ds(base, CHUNK)], si2.at[slot],
                         ssem[slot])
        pltpu.async_copy(dst_hbm.at[pl.ds(base, CHUNK)], di2.at[slot],
                         dwsem[slot])
        pltpu.async_copy(w_hbm.at[pl.ds(base, CHUNK)], wv[slot],
                         dwsem[slot])

      def launch(slot):
        pltpu.make_async_copy(src_hbm.at[pl.ds(0, CHUNK)], si2.at[slot],
                              ssem[slot]).wait()
        pltpu.async_copy(p_hbm.at[si2.at[slot]], rows[slot], gsem[slot])

      def drain(slot, c, refill):
        pltpu.make_async_copy(dst_hbm.at[pl.ds(0, CHUNK)], di2.at[slot],
                              dwsem[slot]).wait()
        pltpu.make_async_copy(w_hbm.at[pl.ds(0, CHUNK)], wv[slot],
                              dwsem[slot]).wait()
        pltpu.make_async_copy(p_hbm.at[pl.ds(0, CHUNK)], rows[slot],
                              gsem[slot]).wait()

        def mul(e, c2):
          for v in range(H // 16):
            sl = pl.ds(v * 16, 16)
            rows[slot][e, sl] = rows[slot][e, sl] * wv[slot][e, sl]
          return c2

        lax.fori_loop(0, CHUNK, mul, 0)
        pltpu.sync_copy(rows[slot], acc_sh.at[di2.at[slot]], add=True)
        if refill:
          nxt = c + 2

          @pl.when(nxt < n_chunk)
          def _():
            fill_idx(slot, nxt)
            launch(slot)

      fill_idx(0, 0)
      launch(0)
      fill_idx(1, 1)
      launch(1)

      def body(p, carry):
        c0 = 2 * p
        drain(0, c0, True)
        drain(1, c0 + 1, True)
        return carry

      lax.fori_loop(0, n_chunk // 2, body, 0)
      if n_chunk % 2:
        drain(0, n_chunk - 1, False)
      plsc.subcore_barrier()

      @pl.when(sid < 10)
      def _():
        pltpu.sync_copy(acc_sh.at[pl.ds(sid * ZROWS, ZROWS)],
                        out_hbm.at[cid, pl.ds(sid * ZROWS, ZROWS)])

    @pl.when(cid == 0)
    def _():
      run(pa_hbm, wa_hbm)

    @pl.when(cid == 1)
    def _():
      run(pb_hbm, wb_hbm)

  return k(p_a, p_b, w_a, w_b, src, dst, zrows)


def _full(shape):
  return pl.BlockSpec(shape, lambda i: (0,) * len(shape))


def kernel(X, batch, Q, edge_index, edge_weight, edge_attr, ln_g, ln_b,
           mlp_w1, mlp_b1, mlp_w2, mlp_b2, ws1, bs1, ws2, bs2, ws3, bs3,
           wt0, wt1, wt2, wt3, wt4, wt5):
  X2 = X.reshape(N, 9 * H)
  batch2 = batch.astype(jnp.int32).reshape(N, 1)
  Q2 = Q.reshape(N, 1)
  ei = edge_index.astype(jnp.int32)
  dst, src = ei[0], ei[1]

  gn = N // BN
  nspec = pl.BlockSpec((BN, 9 * H), lambda i: (i, 0))
  bspec = pl.BlockSpec((BN, 1), lambda i: (i, 0))
  q16 = pl.BlockSpec((BN, QDIM), lambda i: (i, 0))

  c0, fu, qu, fuq = pl.pallas_call(
      _k_charge,
      grid=(gn,),
      in_specs=[nspec, bspec, _full((1, 3 * H)), _full((1, 3 * H)),
                _full((3 * H, H)), _full((1, H)),
                _full((H, 2 * QDIM)), _full((1, 2 * QDIM))],
      out_specs=[q16, q16, _full((NMOL, QDIM)), _full((NMOL, QDIM))],
      out_shape=[jax.ShapeDtypeStruct((N, QDIM), _f32),
                 jax.ShapeDtypeStruct((N, QDIM), _f32),
                 jax.ShapeDtypeStruct((NMOL, QDIM), _f32),
                 jax.ShapeDtypeStruct((NMOL, QDIM), _f32)],
  )(X2, batch2, ln_g.reshape(1, -1), ln_b.reshape(1, -1),
    mlp_w1, mlp_b1.reshape(1, -1), mlp_w2, mlp_b2.reshape(1, -1))

  charges = pl.pallas_call(
      _k_qeq,
      grid=(gn,),
      in_specs=[q16, q16, bspec, bspec,
                _full((NMOL, QDIM)), _full((NMOL, QDIM))],
      out_specs=q16,
      out_shape=jax.ShapeDtypeStruct((N, QDIM), _f32),
  )(c0, fu, batch2, Q2, qu, fuq)

  idx_all = jnp.concatenate([dst, src])
  cpad = jnp.pad(charges, ((0, 0), (0, H - QDIM)))
  cc = _sc_gather_rows(cpad, idx_all)

  ge = E // BE
  espec = lambda w: pl.BlockSpec((BE, w), lambda i: (i, 0))
  cjspec = pl.BlockSpec((BE, H), lambda i: (i + ge, 0))
  w0, w1, w2 = pl.pallas_call(
      _k_edge,
      grid=(ge,),
      in_specs=[espec(NUM_RBF), espec(H), cjspec, espec(1),
                _full((NUM_RBF + 2 * QDIM, H)), _full((1, H)),
                _full((H, 2 * H)), _full((1, 2 * H)),
                _full((2 * H, 3 * H)), _full((1, 3 * H))],
      out_specs=[espec(H), espec(H), espec(H)],
      out_shape=[jax.ShapeDtypeStruct((E, H), _f32)] * 3,
  )(edge_attr, cc, cc, edge_weight.reshape(E, 1),
    ws1, bs1.reshape(1, -1), ws2, bs2.reshape(1, -1), ws3, bs3.reshape(1, -1))

  hspec = pl.BlockSpec((BN, H), lambda i: (i, 0))
  p_list = pl.pallas_call(
      _k_pack,
      grid=(gn,),
      in_specs=[nspec, _full((H, H)), _full((H, H)), _full((H, H))],
      out_specs=[hspec] * 9,
      out_shape=[jax.ShapeDtypeStruct((N, H), _f32)] * 9,
  )(X2, wt0, wt1, wt2)

  wsel = [w0, w1, w1, w1, w2, w2, w2, w2, w2]
  zrows = jnp.zeros((ZROWS, H), _f32)
  m_list = [None] * 9
  for ca in range(0, 9, 2):
    cb = min(ca + 1, 8)
    out2 = _sc_message(p_list[ca], p_list[cb], wsel[ca], wsel[cb],
                       src, dst, zrows)
    m_list[ca] = out2[0]
    if cb != ca:
      m_list[cb] = out2[1]

  xo2 = pl.pallas_call(
      _k_final,
      grid=(gn,),
      in_specs=[nspec, _full((H, H)), _full((H, H)), _full((H, H))]
               + [hspec] * 18,
      out_specs=nspec,
      out_shape=jax.ShapeDtypeStruct((N, 9 * H), _f32),
  )(X2, wt3, wt4, wt5, *p_list, *m_list)

  return xo2.reshape(N, 3, 3, H)
